# baseline (speedup 1.0000x reference)
import dataclasses
import functools

import jax
import jax.numpy as jnp
from jax import lax
from jax.experimental import pallas as pl
from jax.experimental.pallas import tpu as pltpu
from jax.experimental.pallas import tpu_sc as plsc

N = 10000
N_GENES = 2048
DIM = 128
HID = 512
E = 160000

NC = 2
NS = 16
LANES = 16
NW = NC * NS
CHUNK = 80
E_PAD = 163840
TOTAL_CHUNKS = E_PAD // CHUNK
C0 = 128
C1 = 0
NBUF = 3
SLAB = 200
NSLAB = N // SLAB

BLK_A = 400
BLK_E = 400
BLK_C = 1000


def _erf(x):
    return lax.erf(x)


def _encoder_block(x_bf, W1, b1, g, be, W2):
    h = jnp.dot(x_bf, W1, preferred_element_type=jnp.float32) + b1
    mu = jnp.mean(h, axis=-1, keepdims=True)
    var = jnp.mean((h - mu) ** 2, axis=-1, keepdims=True)
    h = (h - mu) * lax.rsqrt(var + 1e-5) * g + be
    h = 0.5 * h * (1.0 + _erf(h * 0.7071067811865476))
    return jnp.dot(h.astype(jnp.bfloat16), W2, preferred_element_type=jnp.float32)


def _enc_kernel(rna_ref, ribo_ref, W1r_ref, b1r_ref, g1r_ref, be1r_ref,
                W2r_ref, b2r_ref, W1b_ref, b1b_ref, g1b_ref, be1b_ref,
                W2b_ref, b2b_ref, Wf_ref, bf_ref, Wg1_ref, out_ref):
    z_rna = _encoder_block(rna_ref[...], W1r_ref[...], b1r_ref[...],
                           g1r_ref[...], be1r_ref[...], W2r_ref[...]) + b2r_ref[...]
    z_ribo = _encoder_block(ribo_ref[...], W1b_ref[...], b1b_ref[...],
                            g1b_ref[...], be1b_ref[...], W2b_ref[...]) + b2b_ref[...]
    wf = Wf_ref[...].reshape(1, DIM)
    s_rna = jnp.sum(jnp.tanh(z_rna) * wf, axis=-1, keepdims=True) + bf_ref[...]
    s_ribo = jnp.sum(jnp.tanh(z_ribo) * wf, axis=-1, keepdims=True) + bf_ref[...]
    m = jnp.maximum(s_rna, s_ribo)
    e_rna = jnp.exp(s_rna - m)
    e_ribo = jnp.exp(s_ribo - m)
    denom = e_rna + e_ribo
    z = (e_rna * z_rna + e_ribo * z_ribo) / denom
    out_ref[...] = jnp.dot(z.astype(jnp.bfloat16), Wg1_ref[...],
                           preferred_element_type=jnp.float32)


def _run_encoder(rna_bf, ribo_bf, W1r, b1r, g1r, be1r, W2r, b2r,
                 W1b, b1b, g1b, be1b, W2b, b2b, Wf, bf, Wg1):
    nblk = N // BLK_A
    row_spec = pl.BlockSpec((BLK_A, N_GENES), lambda i: (i, 0))
    full = lambda arr: pl.BlockSpec(arr.shape, lambda i: (0,) * arr.ndim)
    return pl.pallas_call(
        _enc_kernel,
        grid=(nblk,),
        in_specs=[row_spec, row_spec] + [full(a) for a in (
            W1r, b1r, g1r, be1r, W2r, b2r, W1b, b1b, g1b, be1b, W2b, b2b,
            Wf, bf, Wg1)],
        out_specs=pl.BlockSpec((BLK_A, DIM), lambda i: (i, 0)),
        out_shape=jax.ShapeDtypeStruct((N, DIM), jnp.float32),
    )(rna_bf, ribo_bf, W1r, b1r, g1r, be1r, W2r, b2r,
      W1b, b1b, g1b, be1b, W2b, b2b, Wf, bf, Wg1)


def _gcn1_kernel(p_ref, bg1_ref, Wg2_ref, out_ref):
    t = jax.nn.relu(p_ref[0] + p_ref[1] + bg1_ref[...])
    out_ref[...] = jnp.dot(t.astype(jnp.bfloat16), Wg2_ref[...],
                           preferred_element_type=jnp.float32)


def _run_gcn1(partials, bg1, Wg2_bf):
    nblk = N // BLK_C
    return pl.pallas_call(
        _gcn1_kernel,
        grid=(nblk,),
        in_specs=[pl.BlockSpec((2, BLK_C, DIM), lambda i: (0, i, 0)),
                  pl.BlockSpec(bg1.shape, lambda i: (0,)),
                  pl.BlockSpec(Wg2_bf.shape, lambda i: (0, 0))],
        out_specs=pl.BlockSpec((BLK_C, DIM), lambda i: (i, 0)),
        out_shape=jax.ShapeDtypeStruct((N, DIM), jnp.float32),
    )(partials, bg1, Wg2_bf)


def _bias_kernel(p_ref, b_ref, out_ref):
    out_ref[...] = p_ref[0] + p_ref[1] + b_ref[...]


def _run_bias(partials, b):
    nblk = N // BLK_C
    return pl.pallas_call(
        _bias_kernel,
        grid=(nblk,),
        in_specs=[pl.BlockSpec((2, BLK_C, DIM), lambda i: (0, i, 0)),
                  pl.BlockSpec(b.shape, lambda i: (0,))],
        out_specs=pl.BlockSpec((BLK_C, DIM), lambda i: (i, 0)),
        out_shape=jax.ShapeDtypeStruct((N, DIM), jnp.float32),
    )(partials, b)


def _dec_kernel(p_ref, Wdr_ref, Wdb_ref, rna_ref, ribo_ref):
    ah = (p_ref[0] + p_ref[1]).astype(jnp.bfloat16)
    rna_ref[...] = jnp.dot(ah, Wdr_ref[...], preferred_element_type=jnp.float32)
    ribo_ref[...] = jnp.dot(ah, Wdb_ref[...], preferred_element_type=jnp.float32)


def _run_decoder(partials, Wdr_bf, Wdb_bf):
    nblk = N // BLK_E
    return pl.pallas_call(
        _dec_kernel,
        grid=(nblk,),
        in_specs=[pl.BlockSpec((2, BLK_E, DIM), lambda i: (0, i, 0)),
                  pl.BlockSpec(Wdr_bf.shape, lambda i: (0, 0)),
                  pl.BlockSpec(Wdb_bf.shape, lambda i: (0, 0))],
        out_specs=[pl.BlockSpec((BLK_E, N_GENES), lambda i: (i, 0)),
                   pl.BlockSpec((BLK_E, N_GENES), lambda i: (i, 0))],
        out_shape=[jax.ShapeDtypeStruct((N, N_GENES), jnp.float32),
                   jax.ShapeDtypeStruct((N, N_GENES), jnp.float32)],
    )(partials, Wdr_bf, Wdb_bf)


def _spmm_body(x_hbm, col_hbm, row_hbm, w_hbm, out_hbm, accum,
               col_v, wb0, wb1, wb2, rows0, rows1, rows2, ridx0, ridx1, ridx2,
               sg0, sg1, sg2, sr0, sr1, sr2, ss0, ss1, ss2):
    c = lax.axis_index("c")
    s = lax.axis_index("s")
    bufs = (rows0, rows1, rows2)
    wbuf = (wb0, wb1, wb2)
    ridx = (ridx0, ridx1, ridx2)
    sg = (sg0, sg1, sg2)
    sr = (sr0, sr1, sr2)
    ss = (ss0, ss1, ss2)

    @pl.loop(0, CHUNK)
    def _zrow(r):
        for q in range(DIM // LANES):
            rows0[r, pl.ds(q * LANES, LANES)] = jnp.zeros((LANES,), jnp.float32)

    nslab = N // CHUNK
    for k in range((nslab + NS - 1) // NS):
        ch = s + k * NS

        @pl.when(ch < nslab)
        def _():
            off = pl.multiple_of(ch * CHUNK, 8)
            pltpu.sync_copy(rows0, accum.at[pl.ds(off, CHUNK)])
    plsc.subcore_barrier()

    def pipeline(base, nchunks):
        if nchunks == 0:
            return
        pltpu.sync_copy(col_hbm.at[pl.ds(base, nchunks)],
                        col_v.at[pl.ds(0, nchunks)])

        def issue(jj, b):
            pltpu.async_copy(row_hbm.at[base + jj], ridx[b], sr[b])
            pltpu.async_copy(w_hbm.at[base + jj], wbuf[b], sr[b])
            pltpu.async_copy(x_hbm.at[col_v.at[jj]], bufs[b], sg[b])

        def wait_issue(jj, b):
            pltpu.make_async_copy(row_hbm.at[base + jj], ridx[b], sr[b]).wait()
            pltpu.make_async_copy(w_hbm.at[base + jj], wbuf[b], sr[b]).wait()
            pltpu.make_async_copy(x_hbm.at[col_v.at[jj]], bufs[b], sg[b]).wait()

        def scale(jj, b):
            buf = bufs[b]

            @pl.loop(0, CHUNK, step=4)
            def _edge(l0):
                for dl in range(4):
                    lv = jnp.full((LANES,), l0 + dl, jnp.int32)
                    wb = plsc.load_gather(wbuf[b], [lv])
                    for q in range(DIM // LANES):
                        sl = pl.ds(q * LANES, LANES)
                        buf[l0 + dl, sl] = buf[l0 + dl, sl] * wb

        def section(jp, b, refill=True):
            jj = jp * NBUF + b
            wait_issue(jj, b)
            scale(jj, b)
            pltpu.async_copy(bufs[b], accum.at[ridx[b]], ss[b], add=True)
            if refill:
                nxt = jj + NBUF

                @pl.when(nxt < nchunks)
                def _():
                    pltpu.make_async_copy(bufs[b], accum.at[ridx[b]],
                                          ss[b]).wait()
                    issue(nxt, b)

        for b in range(NBUF):
            issue(b, b)

        @pl.loop(0, nchunks // NBUF)
        def _pipe(jp):
            for b in range(NBUF):
                section(jp, b)

        for b in range(nchunks % NBUF):
            section(nchunks // NBUF, b, refill=False)
        for b in range(NBUF):
            pltpu.make_async_copy(bufs[b], accum.at[ridx[b]], ss[b]).wait()

    @pl.when(c == 0)
    def _():
        pipeline(s * C0, C0)

    @pl.when(c == 1)
    def _():
        pipeline(NS * C0 + s * C1, C1)

    plsc.subcore_barrier()

    for k in range((NSLAB + NS - 1) // NS):
        slab = s + k * NS

        @pl.when(slab < NSLAB)
        def _():
            off = pl.multiple_of(slab * SLAB, 8)
            pltpu.sync_copy(accum.at[pl.ds(off, SLAB)],
                            out_hbm.at[c, pl.ds(off, SLAB)])


@jax.jit
def _spmm(x, col3, row3, w3):
    mesh = plsc.VectorSubcoreMesh(core_axis_name="c", subcore_axis_name="s")
    cp = pltpu.CompilerParams()
    if "needs_layout_passes" in pltpu.CompilerParams.__dataclass_fields__:
        cp = dataclasses.replace(cp, needs_layout_passes=False)
    kern = pl.kernel(
        _spmm_body,
        out_type=jax.ShapeDtypeStruct((NC, N, DIM), jnp.float32),
        mesh=mesh,
        scratch_types=[
            pltpu.VMEM_SHARED((N, DIM), jnp.float32),
            pltpu.VMEM((C0, CHUNK), jnp.int32),
            pltpu.VMEM((CHUNK,), jnp.float32),
            pltpu.VMEM((CHUNK,), jnp.float32),
            pltpu.VMEM((CHUNK,), jnp.float32),
            pltpu.VMEM((CHUNK, DIM), jnp.float32),
            pltpu.VMEM((CHUNK, DIM), jnp.float32),
            pltpu.VMEM((CHUNK, DIM), jnp.float32),
            pltpu.VMEM((CHUNK,), jnp.int32),
            pltpu.VMEM((CHUNK,), jnp.int32),
            pltpu.VMEM((CHUNK,), jnp.int32),
            pltpu.SemaphoreType.DMA,
            pltpu.SemaphoreType.DMA,
            pltpu.SemaphoreType.DMA,
            pltpu.SemaphoreType.DMA,
            pltpu.SemaphoreType.DMA,
            pltpu.SemaphoreType.DMA,
            pltpu.SemaphoreType.DMA,
            pltpu.SemaphoreType.DMA,
            pltpu.SemaphoreType.DMA,
        ],
        compiler_params=cp,
    )
    return kern(x, col3, row3, w3)


def kernel(rna_expr, ribo_expr, edge_index, edge_weight, W1_rna, b1_rna,
           g1_rna, be1_rna, W2_rna, b2_rna, W1_ribo, b1_ribo, g1_ribo,
           be1_ribo, W2_ribo, b2_ribo, Wf, bf, Wg1, bg1, Wg2, bg2,
           Wd_rna, Wd_ribo):
    f32 = jnp.float32
    bf16 = jnp.bfloat16

    rna_bf = rna_expr.astype(bf16)
    ribo_bf = ribo_expr.astype(bf16)
    W1r = W1_rna.astype(bf16)
    W2r = W2_rna.astype(bf16)
    W1b = W1_ribo.astype(bf16)
    W2b = W2_ribo.astype(bf16)
    Wg1_bf = Wg1.astype(bf16)
    Wg2_bf = Wg2.astype(bf16)
    Wdr_bf = Wd_rna.astype(bf16)
    Wdb_bf = Wd_ribo.astype(bf16)

    pad = E_PAD - E
    col3 = jnp.concatenate(
        [edge_index[1], jnp.zeros((pad,), jnp.int32)]).reshape(TOTAL_CHUNKS, CHUNK)
    row3 = jnp.concatenate(
        [edge_index[0], jnp.zeros((pad,), jnp.int32)]).reshape(TOTAL_CHUNKS, CHUNK)
    w3 = jnp.concatenate(
        [edge_weight, jnp.zeros((pad,), f32)]).reshape(TOTAL_CHUNKS, CHUNK)

    y1 = _run_encoder(rna_bf, ribo_bf, W1r, b1_rna, g1_rna, be1_rna, W2r,
                      b2_rna, W1b, b1_ribo, g1_ribo, be1_ribo, W2b, b2_ribo,
                      Wf, bf, Wg1_bf)
    s1 = _spmm(y1, col3, row3, w3)
    y2 = _run_gcn1(s1, bg1, Wg2_bf)
    s2 = _spmm(y2, col3, row3, w3)
    h = _run_bias(s2, bg2)
    s3 = _spmm(h, col3, row3, w3)
    recon_rna, recon_ribo = _run_decoder(s3, Wdr_bf, Wdb_bf)
    return (h, recon_rna, recon_ribo)


# speedup vs baseline: 1.3387x; 1.3387x over previous
import dataclasses
import functools

import jax
import jax.numpy as jnp
from jax import lax
from jax.experimental import pallas as pl
from jax.experimental.pallas import tpu as pltpu
from jax.experimental.pallas import tpu_sc as plsc

N = 10000
N_GENES = 2048
DIM = 128
HID = 512
E = 160000

NC = 2
NS = 16
LANES = 16
NW = NC * NS
CHUNK = 80
E_PAD = 163840
TOTAL_CHUNKS = E_PAD // CHUNK
C0 = 112
C1 = 16
NBUF = 3
SLAB = 200
NSLAB = N // SLAB

BLK_A = 400
BLK_E = 400
BLK_C = 1000


def _erf(x):
    return lax.erf(x)


def _encoder_block(x_bf, W1, b1, g, be, W2):
    h = jnp.dot(x_bf, W1, preferred_element_type=jnp.float32) + b1
    mu = jnp.mean(h, axis=-1, keepdims=True)
    var = jnp.mean((h - mu) ** 2, axis=-1, keepdims=True)
    h = (h - mu) * lax.rsqrt(var + 1e-5) * g + be
    h = 0.5 * h * (1.0 + _erf(h * 0.7071067811865476))
    return jnp.dot(h.astype(jnp.bfloat16), W2, preferred_element_type=jnp.float32)


def _enc_kernel(rna_ref, ribo_ref, W1r_ref, b1r_ref, g1r_ref, be1r_ref,
                W2r_ref, b2r_ref, W1b_ref, b1b_ref, g1b_ref, be1b_ref,
                W2b_ref, b2b_ref, Wf_ref, bf_ref, Wg1_ref, out_ref):
    z_rna = _encoder_block(rna_ref[...], W1r_ref[...], b1r_ref[...],
                           g1r_ref[...], be1r_ref[...], W2r_ref[...]) + b2r_ref[...]
    z_ribo = _encoder_block(ribo_ref[...], W1b_ref[...], b1b_ref[...],
                            g1b_ref[...], be1b_ref[...], W2b_ref[...]) + b2b_ref[...]
    wf = Wf_ref[...].reshape(1, DIM)
    s_rna = jnp.sum(jnp.tanh(z_rna) * wf, axis=-1, keepdims=True) + bf_ref[...]
    s_ribo = jnp.sum(jnp.tanh(z_ribo) * wf, axis=-1, keepdims=True) + bf_ref[...]
    m = jnp.maximum(s_rna, s_ribo)
    e_rna = jnp.exp(s_rna - m)
    e_ribo = jnp.exp(s_ribo - m)
    denom = e_rna + e_ribo
    z = (e_rna * z_rna + e_ribo * z_ribo) / denom
    out_ref[...] = jnp.dot(z.astype(jnp.bfloat16), Wg1_ref[...],
                           preferred_element_type=jnp.float32)


def _run_encoder(rna_bf, ribo_bf, W1r, b1r, g1r, be1r, W2r, b2r,
                 W1b, b1b, g1b, be1b, W2b, b2b, Wf, bf, Wg1):
    nblk = N // BLK_A
    row_spec = pl.BlockSpec((BLK_A, N_GENES), lambda i: (i, 0))
    full = lambda arr: pl.BlockSpec(arr.shape, lambda i: (0,) * arr.ndim)
    return pl.pallas_call(
        _enc_kernel,
        grid=(nblk,),
        in_specs=[row_spec, row_spec] + [full(a) for a in (
            W1r, b1r, g1r, be1r, W2r, b2r, W1b, b1b, g1b, be1b, W2b, b2b,
            Wf, bf, Wg1)],
        out_specs=pl.BlockSpec((BLK_A, DIM), lambda i: (i, 0)),
        out_shape=jax.ShapeDtypeStruct((N, DIM), jnp.float32),
    )(rna_bf, ribo_bf, W1r, b1r, g1r, be1r, W2r, b2r,
      W1b, b1b, g1b, be1b, W2b, b2b, Wf, bf, Wg1)


def _gcn1_kernel(p_ref, bg1_ref, Wg2_ref, out_ref):
    t = jax.nn.relu(p_ref[0] + p_ref[1] + bg1_ref[...])
    out_ref[...] = jnp.dot(t.astype(jnp.bfloat16), Wg2_ref[...],
                           preferred_element_type=jnp.float32)


def _run_gcn1(partials, bg1, Wg2_bf):
    nblk = N // BLK_C
    return pl.pallas_call(
        _gcn1_kernel,
        grid=(nblk,),
        in_specs=[pl.BlockSpec((2, BLK_C, DIM), lambda i: (0, i, 0)),
                  pl.BlockSpec(bg1.shape, lambda i: (0,)),
                  pl.BlockSpec(Wg2_bf.shape, lambda i: (0, 0))],
        out_specs=pl.BlockSpec((BLK_C, DIM), lambda i: (i, 0)),
        out_shape=jax.ShapeDtypeStruct((N, DIM), jnp.float32),
    )(partials, bg1, Wg2_bf)


def _bias_kernel(p_ref, b_ref, out_ref):
    out_ref[...] = p_ref[0] + p_ref[1] + b_ref[...]


def _run_bias(partials, b):
    nblk = N // BLK_C
    return pl.pallas_call(
        _bias_kernel,
        grid=(nblk,),
        in_specs=[pl.BlockSpec((2, BLK_C, DIM), lambda i: (0, i, 0)),
                  pl.BlockSpec(b.shape, lambda i: (0,))],
        out_specs=pl.BlockSpec((BLK_C, DIM), lambda i: (i, 0)),
        out_shape=jax.ShapeDtypeStruct((N, DIM), jnp.float32),
    )(partials, b)


def _dec_kernel(p_ref, Wdr_ref, Wdb_ref, rna_ref, ribo_ref):
    ah = (p_ref[0] + p_ref[1]).astype(jnp.bfloat16)
    rna_ref[...] = jnp.dot(ah, Wdr_ref[...], preferred_element_type=jnp.float32)
    ribo_ref[...] = jnp.dot(ah, Wdb_ref[...], preferred_element_type=jnp.float32)


def _run_decoder(partials, Wdr_bf, Wdb_bf):
    nblk = N // BLK_E
    return pl.pallas_call(
        _dec_kernel,
        grid=(nblk,),
        in_specs=[pl.BlockSpec((2, BLK_E, DIM), lambda i: (0, i, 0)),
                  pl.BlockSpec(Wdr_bf.shape, lambda i: (0, 0)),
                  pl.BlockSpec(Wdb_bf.shape, lambda i: (0, 0))],
        out_specs=[pl.BlockSpec((BLK_E, N_GENES), lambda i: (i, 0)),
                   pl.BlockSpec((BLK_E, N_GENES), lambda i: (i, 0))],
        out_shape=[jax.ShapeDtypeStruct((N, N_GENES), jnp.float32),
                   jax.ShapeDtypeStruct((N, N_GENES), jnp.float32)],
    )(partials, Wdr_bf, Wdb_bf)


def _spmm_body(x_hbm, col_hbm, row_hbm, w_hbm, out_hbm, accum,
               col_v, wb0, wb1, wb2, rows0, rows1, rows2, ridx0, ridx1, ridx2,
               sg0, sg1, sg2, sr0, sr1, sr2, ss0, ss1, ss2):
    c = lax.axis_index("c")
    s = lax.axis_index("s")
    bufs = (rows0, rows1, rows2)
    wbuf = (wb0, wb1, wb2)
    ridx = (ridx0, ridx1, ridx2)
    sg = (sg0, sg1, sg2)
    sr = (sr0, sr1, sr2)
    ss = (ss0, ss1, ss2)

    @pl.loop(0, CHUNK)
    def _zrow(r):
        for q in range(DIM // LANES):
            rows0[r, pl.ds(q * LANES, LANES)] = jnp.zeros((LANES,), jnp.float32)

    nslab = N // CHUNK
    for k in range((nslab + NS - 1) // NS):
        ch = s + k * NS

        @pl.when(ch < nslab)
        def _():
            off = pl.multiple_of(ch * CHUNK, 8)
            pltpu.sync_copy(rows0, accum.at[pl.ds(off, CHUNK)])
    plsc.subcore_barrier()

    def pipeline(base, nchunks):
        if nchunks == 0:
            return
        pltpu.sync_copy(col_hbm.at[pl.ds(base, nchunks)],
                        col_v.at[pl.ds(0, nchunks)])

        def issue(jj, b):
            pltpu.async_copy(row_hbm.at[base + jj], ridx[b], sr[b])
            pltpu.async_copy(w_hbm.at[base + jj], wbuf[b], sr[b])
            pltpu.async_copy(x_hbm.at[col_v.at[jj]], bufs[b], sg[b])

        def wait_issue(jj, b):
            pltpu.make_async_copy(row_hbm.at[base + jj], ridx[b], sr[b]).wait()
            pltpu.make_async_copy(w_hbm.at[base + jj], wbuf[b], sr[b]).wait()
            pltpu.make_async_copy(x_hbm.at[col_v.at[jj]], bufs[b], sg[b]).wait()

        def scale(jj, b):
            buf = bufs[b]

            @pl.loop(0, CHUNK, step=4)
            def _edge(l0):
                for dl in range(4):
                    lv = jnp.full((LANES,), l0 + dl, jnp.int32)
                    wb = plsc.load_gather(wbuf[b], [lv])
                    for q in range(DIM // LANES):
                        sl = pl.ds(q * LANES, LANES)
                        buf[l0 + dl, sl] = buf[l0 + dl, sl] * wb

        def section(jp, b, refill=True):
            jj = jp * NBUF + b
            wait_issue(jj, b)
            scale(jj, b)
            pltpu.async_copy(bufs[b], accum.at[ridx[b]], ss[b], add=True)
            if refill:
                nxt = jj + NBUF

                @pl.when(nxt < nchunks)
                def _():
                    pltpu.make_async_copy(bufs[b], accum.at[ridx[b]],
                                          ss[b]).wait()
                    issue(nxt, b)

        for b in range(NBUF):
            issue(b, b)

        @pl.loop(0, nchunks // NBUF)
        def _pipe(jp):
            for b in range(NBUF):
                section(jp, b)

        for b in range(nchunks % NBUF):
            section(nchunks // NBUF, b, refill=False)
        for b in range(NBUF):
            pltpu.make_async_copy(bufs[b], accum.at[ridx[b]], ss[b]).wait()

    @pl.when(c == 0)
    def _():
        pipeline(s * C0, C0)

    @pl.when(c == 1)
    def _():
        pipeline(NS * C0 + s * C1, C1)

    plsc.subcore_barrier()

    for k in range((NSLAB + NS - 1) // NS):
        slab = s + k * NS

        @pl.when(slab < NSLAB)
        def _():
            off = pl.multiple_of(slab * SLAB, 8)
            pltpu.sync_copy(accum.at[pl.ds(off, SLAB)],
                            out_hbm.at[c, pl.ds(off, SLAB)])


@jax.jit
def _spmm(x, col3, row3, w3):
    mesh = plsc.VectorSubcoreMesh(core_axis_name="c", subcore_axis_name="s")
    cp = pltpu.CompilerParams()
    if "needs_layout_passes" in pltpu.CompilerParams.__dataclass_fields__:
        cp = dataclasses.replace(cp, needs_layout_passes=False)
    kern = pl.kernel(
        _spmm_body,
        out_type=jax.ShapeDtypeStruct((NC, N, DIM), jnp.float32),
        mesh=mesh,
        scratch_types=[
            pltpu.VMEM_SHARED((N, DIM), jnp.float32),
            pltpu.VMEM((C0, CHUNK), jnp.int32),
            pltpu.VMEM((CHUNK,), jnp.float32),
            pltpu.VMEM((CHUNK,), jnp.float32),
            pltpu.VMEM((CHUNK,), jnp.float32),
            pltpu.VMEM((CHUNK, DIM), jnp.float32),
            pltpu.VMEM((CHUNK, DIM), jnp.float32),
            pltpu.VMEM((CHUNK, DIM), jnp.float32),
            pltpu.VMEM((CHUNK,), jnp.int32),
            pltpu.VMEM((CHUNK,), jnp.int32),
            pltpu.VMEM((CHUNK,), jnp.int32),
            pltpu.SemaphoreType.DMA,
            pltpu.SemaphoreType.DMA,
            pltpu.SemaphoreType.DMA,
            pltpu.SemaphoreType.DMA,
            pltpu.SemaphoreType.DMA,
            pltpu.SemaphoreType.DMA,
            pltpu.SemaphoreType.DMA,
            pltpu.SemaphoreType.DMA,
            pltpu.SemaphoreType.DMA,
        ],
        compiler_params=cp,
    )
    return kern(x, col3, row3, w3)


def kernel(rna_expr, ribo_expr, edge_index, edge_weight, W1_rna, b1_rna,
           g1_rna, be1_rna, W2_rna, b2_rna, W1_ribo, b1_ribo, g1_ribo,
           be1_ribo, W2_ribo, b2_ribo, Wf, bf, Wg1, bg1, Wg2, bg2,
           Wd_rna, Wd_ribo):
    f32 = jnp.float32
    bf16 = jnp.bfloat16

    rna_bf = rna_expr.astype(bf16)
    ribo_bf = ribo_expr.astype(bf16)
    W1r = W1_rna.astype(bf16)
    W2r = W2_rna.astype(bf16)
    W1b = W1_ribo.astype(bf16)
    W2b = W2_ribo.astype(bf16)
    Wg1_bf = Wg1.astype(bf16)
    Wg2_bf = Wg2.astype(bf16)
    Wdr_bf = Wd_rna.astype(bf16)
    Wdb_bf = Wd_ribo.astype(bf16)

    pad = E_PAD - E
    col3 = jnp.concatenate(
        [edge_index[1], jnp.zeros((pad,), jnp.int32)]).reshape(TOTAL_CHUNKS, CHUNK)
    row3 = jnp.concatenate(
        [edge_index[0], jnp.zeros((pad,), jnp.int32)]).reshape(TOTAL_CHUNKS, CHUNK)
    w3 = jnp.concatenate(
        [edge_weight, jnp.zeros((pad,), f32)]).reshape(TOTAL_CHUNKS, CHUNK)

    y1 = _run_encoder(rna_bf, ribo_bf, W1r, b1_rna, g1_rna, be1_rna, W2r,
                      b2_rna, W1b, b1_ribo, g1_ribo, be1_ribo, W2b, b2_ribo,
                      Wf, bf, Wg1_bf)
    s1 = _spmm(y1, col3, row3, w3)
    y2 = _run_gcn1(s1, bg1, Wg2_bf)
    s2 = _spmm(y2, col3, row3, w3)
    h = _run_bias(s2, bg2)
    s3 = _spmm(h, col3, row3, w3)
    recon_rna, recon_ribo = _run_decoder(s3, Wdr_bf, Wdb_bf)
    return (h, recon_rna, recon_ribo)


# speedup vs baseline: 1.3677x; 1.0217x over previous
import dataclasses
import functools

import jax
import jax.numpy as jnp
from jax import lax
from jax.experimental import pallas as pl
from jax.experimental.pallas import tpu as pltpu
from jax.experimental.pallas import tpu_sc as plsc

N = 10000
N_GENES = 2048
DIM = 128
HID = 512
E = 160000

NC = 2
NS = 16
LANES = 16
NW = NC * NS
CHUNK = 80
E_PAD = 163840
TOTAL_CHUNKS = E_PAD // CHUNK
C0 = 112
C1 = 16
NBUF = 3
SLAB = 200
NSLAB = N // SLAB

BLK_A = 1000
BLK_E = 1000
BLK_C = 1000


def _erf(x):
    return lax.erf(x)


def _encoder_block(x_bf, W1, b1, g, be, W2):
    h = jnp.dot(x_bf, W1, preferred_element_type=jnp.float32) + b1
    mu = jnp.mean(h, axis=-1, keepdims=True)
    var = jnp.mean((h - mu) ** 2, axis=-1, keepdims=True)
    h = (h - mu) * lax.rsqrt(var + 1e-5) * g + be
    h = 0.5 * h * (1.0 + _erf(h * 0.7071067811865476))
    return jnp.dot(h.astype(jnp.bfloat16), W2, preferred_element_type=jnp.float32)


def _enc_kernel(rna_ref, ribo_ref, W1r_ref, b1r_ref, g1r_ref, be1r_ref,
                W2r_ref, b2r_ref, W1b_ref, b1b_ref, g1b_ref, be1b_ref,
                W2b_ref, b2b_ref, Wf_ref, bf_ref, Wg1_ref, out_ref):
    z_rna = _encoder_block(rna_ref[...], W1r_ref[...], b1r_ref[...],
                           g1r_ref[...], be1r_ref[...], W2r_ref[...]) + b2r_ref[...]
    z_ribo = _encoder_block(ribo_ref[...], W1b_ref[...], b1b_ref[...],
                            g1b_ref[...], be1b_ref[...], W2b_ref[...]) + b2b_ref[...]
    wf = Wf_ref[...].reshape(1, DIM)
    s_rna = jnp.sum(jnp.tanh(z_rna) * wf, axis=-1, keepdims=True) + bf_ref[...]
    s_ribo = jnp.sum(jnp.tanh(z_ribo) * wf, axis=-1, keepdims=True) + bf_ref[...]
    m = jnp.maximum(s_rna, s_ribo)
    e_rna = jnp.exp(s_rna - m)
    e_ribo = jnp.exp(s_ribo - m)
    denom = e_rna + e_ribo
    z = (e_rna * z_rna + e_ribo * z_ribo) / denom
    out_ref[...] = jnp.dot(z.astype(jnp.bfloat16), Wg1_ref[...],
                           preferred_element_type=jnp.float32)


def _run_encoder(rna_bf, ribo_bf, W1r, b1r, g1r, be1r, W2r, b2r,
                 W1b, b1b, g1b, be1b, W2b, b2b, Wf, bf, Wg1):
    nblk = N // BLK_A
    row_spec = pl.BlockSpec((BLK_A, N_GENES), lambda i: (i, 0))
    full = lambda arr: pl.BlockSpec(arr.shape, lambda i: (0,) * arr.ndim)
    return pl.pallas_call(
        _enc_kernel,
        grid=(nblk,),
        in_specs=[row_spec, row_spec] + [full(a) for a in (
            W1r, b1r, g1r, be1r, W2r, b2r, W1b, b1b, g1b, be1b, W2b, b2b,
            Wf, bf, Wg1)],
        out_specs=pl.BlockSpec((BLK_A, DIM), lambda i: (i, 0)),
        out_shape=jax.ShapeDtypeStruct((N, DIM), jnp.float32),
    )(rna_bf, ribo_bf, W1r, b1r, g1r, be1r, W2r, b2r,
      W1b, b1b, g1b, be1b, W2b, b2b, Wf, bf, Wg1)


def _gcn1_kernel(p_ref, bg1_ref, Wg2_ref, out_ref):
    t = jax.nn.relu(p_ref[0] + p_ref[1] + bg1_ref[...])
    out_ref[...] = jnp.dot(t.astype(jnp.bfloat16), Wg2_ref[...],
                           preferred_element_type=jnp.float32)


def _run_gcn1(partials, bg1, Wg2_bf):
    nblk = N // BLK_C
    return pl.pallas_call(
        _gcn1_kernel,
        grid=(nblk,),
        in_specs=[pl.BlockSpec((2, BLK_C, DIM), lambda i: (0, i, 0)),
                  pl.BlockSpec(bg1.shape, lambda i: (0,)),
                  pl.BlockSpec(Wg2_bf.shape, lambda i: (0, 0))],
        out_specs=pl.BlockSpec((BLK_C, DIM), lambda i: (i, 0)),
        out_shape=jax.ShapeDtypeStruct((N, DIM), jnp.float32),
    )(partials, bg1, Wg2_bf)


def _bias_kernel(p_ref, b_ref, out_ref):
    out_ref[...] = p_ref[0] + p_ref[1] + b_ref[...]


def _run_bias(partials, b):
    nblk = N // BLK_C
    return pl.pallas_call(
        _bias_kernel,
        grid=(nblk,),
        in_specs=[pl.BlockSpec((2, BLK_C, DIM), lambda i: (0, i, 0)),
                  pl.BlockSpec(b.shape, lambda i: (0,))],
        out_specs=pl.BlockSpec((BLK_C, DIM), lambda i: (i, 0)),
        out_shape=jax.ShapeDtypeStruct((N, DIM), jnp.float32),
    )(partials, b)


def _dec_kernel(p_ref, Wdr_ref, Wdb_ref, rna_ref, ribo_ref):
    ah = (p_ref[0] + p_ref[1]).astype(jnp.bfloat16)
    rna_ref[...] = jnp.dot(ah, Wdr_ref[...], preferred_element_type=jnp.float32)
    ribo_ref[...] = jnp.dot(ah, Wdb_ref[...], preferred_element_type=jnp.float32)


def _run_decoder(partials, Wdr_bf, Wdb_bf):
    nblk = N // BLK_E
    return pl.pallas_call(
        _dec_kernel,
        grid=(nblk,),
        in_specs=[pl.BlockSpec((2, BLK_E, DIM), lambda i: (0, i, 0)),
                  pl.BlockSpec(Wdr_bf.shape, lambda i: (0, 0)),
                  pl.BlockSpec(Wdb_bf.shape, lambda i: (0, 0))],
        out_specs=[pl.BlockSpec((BLK_E, N_GENES), lambda i: (i, 0)),
                   pl.BlockSpec((BLK_E, N_GENES), lambda i: (i, 0))],
        out_shape=[jax.ShapeDtypeStruct((N, N_GENES), jnp.float32),
                   jax.ShapeDtypeStruct((N, N_GENES), jnp.float32)],
    )(partials, Wdr_bf, Wdb_bf)


def _spmm_body(x_hbm, col_hbm, row_hbm, w_hbm, z_hbm, out_hbm, accum,
               col_v, wb0, wb1, wb2, rows0, rows1, rows2, ridx0, ridx1, ridx2,
               sg0, sg1, sg2, sr0, sr1, sr2, ss0, ss1, ss2):
    c = lax.axis_index("c")
    s = lax.axis_index("s")
    bufs = (rows0, rows1, rows2)
    wbuf = (wb0, wb1, wb2)
    ridx = (ridx0, ridx1, ridx2)
    sg = (sg0, sg1, sg2)
    sr = (sr0, sr1, sr2)
    ss = (ss0, ss1, ss2)

    def pipeline(base, nchunks):
        if nchunks == 0:
            plsc.subcore_barrier()
            return
        pltpu.sync_copy(col_hbm.at[pl.ds(base, nchunks)],
                        col_v.at[pl.ds(0, nchunks)])

        def issue(jj, b):
            pltpu.async_copy(row_hbm.at[base + jj], ridx[b], sr[b])
            pltpu.async_copy(w_hbm.at[base + jj], wbuf[b], sr[b])
            pltpu.async_copy(x_hbm.at[col_v.at[jj]], bufs[b], sg[b])

        def wait_issue(jj, b):
            pltpu.make_async_copy(row_hbm.at[base + jj], ridx[b], sr[b]).wait()
            pltpu.make_async_copy(w_hbm.at[base + jj], wbuf[b], sr[b]).wait()
            pltpu.make_async_copy(x_hbm.at[col_v.at[jj]], bufs[b], sg[b]).wait()

        def scale(jj, b):
            buf = bufs[b]

            @pl.loop(0, CHUNK, step=4)
            def _edge(l0):
                for dl in range(4):
                    lv = jnp.full((LANES,), l0 + dl, jnp.int32)
                    wb = plsc.load_gather(wbuf[b], [lv])
                    for q in range(DIM // LANES):
                        sl = pl.ds(q * LANES, LANES)
                        buf[l0 + dl, sl] = buf[l0 + dl, sl] * wb

        def section(jp, b, refill=True):
            jj = jp * NBUF + b
            wait_issue(jj, b)
            scale(jj, b)
            pltpu.async_copy(bufs[b], accum.at[ridx[b]], ss[b], add=True)
            if refill:
                nxt = jj + NBUF

                @pl.when(nxt < nchunks)
                def _():
                    pltpu.make_async_copy(bufs[b], accum.at[ridx[b]],
                                          ss[b]).wait()
                    issue(nxt, b)

        for b in range(NBUF):
            issue(b, b)

        nslab = N // CHUNK
        for k in range((nslab + NS - 1) // NS):
            ch = s + k * NS

            @pl.when(ch < nslab)
            def _():
                off = pl.multiple_of(ch * CHUNK, 8)
                pltpu.sync_copy(z_hbm, accum.at[pl.ds(off, CHUNK)])
        plsc.subcore_barrier()

        @pl.loop(0, nchunks // NBUF)
        def _pipe(jp):
            for b in range(NBUF):
                section(jp, b)

        for b in range(nchunks % NBUF):
            section(nchunks // NBUF, b, refill=False)
        for b in range(NBUF):
            pltpu.make_async_copy(bufs[b], accum.at[ridx[b]], ss[b]).wait()

    @pl.when(c == 0)
    def _():
        pipeline(s * C0, C0)

    @pl.when(c == 1)
    def _():
        pipeline(NS * C0 + s * C1, C1)

    plsc.subcore_barrier()

    for k in range((NSLAB + NS - 1) // NS):
        slab = s + k * NS

        @pl.when(slab < NSLAB)
        def _():
            off = pl.multiple_of(slab * SLAB, 8)
            pltpu.sync_copy(accum.at[pl.ds(off, SLAB)],
                            out_hbm.at[c, pl.ds(off, SLAB)])


@jax.jit
def _spmm(x, col3, row3, w3, zeros2d):
    mesh = plsc.VectorSubcoreMesh(core_axis_name="c", subcore_axis_name="s")
    cp = pltpu.CompilerParams()
    if "needs_layout_passes" in pltpu.CompilerParams.__dataclass_fields__:
        cp = dataclasses.replace(cp, needs_layout_passes=False)
    kern = pl.kernel(
        _spmm_body,
        out_type=jax.ShapeDtypeStruct((NC, N, DIM), jnp.float32),
        mesh=mesh,
        scratch_types=[
            pltpu.VMEM_SHARED((N, DIM), jnp.float32),
            pltpu.VMEM((C0, CHUNK), jnp.int32),
            pltpu.VMEM((CHUNK,), jnp.float32),
            pltpu.VMEM((CHUNK,), jnp.float32),
            pltpu.VMEM((CHUNK,), jnp.float32),
            pltpu.VMEM((CHUNK, DIM), jnp.float32),
            pltpu.VMEM((CHUNK, DIM), jnp.float32),
            pltpu.VMEM((CHUNK, DIM), jnp.float32),
            pltpu.VMEM((CHUNK,), jnp.int32),
            pltpu.VMEM((CHUNK,), jnp.int32),
            pltpu.VMEM((CHUNK,), jnp.int32),
            pltpu.SemaphoreType.DMA,
            pltpu.SemaphoreType.DMA,
            pltpu.SemaphoreType.DMA,
            pltpu.SemaphoreType.DMA,
            pltpu.SemaphoreType.DMA,
            pltpu.SemaphoreType.DMA,
            pltpu.SemaphoreType.DMA,
            pltpu.SemaphoreType.DMA,
            pltpu.SemaphoreType.DMA,
        ],
        compiler_params=cp,
    )
    return kern(x, col3, row3, w3, zeros2d)


def kernel(rna_expr, ribo_expr, edge_index, edge_weight, W1_rna, b1_rna,
           g1_rna, be1_rna, W2_rna, b2_rna, W1_ribo, b1_ribo, g1_ribo,
           be1_ribo, W2_ribo, b2_ribo, Wf, bf, Wg1, bg1, Wg2, bg2,
           Wd_rna, Wd_ribo):
    f32 = jnp.float32
    bf16 = jnp.bfloat16

    rna_bf = rna_expr.astype(bf16)
    ribo_bf = ribo_expr.astype(bf16)
    W1r = W1_rna.astype(bf16)
    W2r = W2_rna.astype(bf16)
    W1b = W1_ribo.astype(bf16)
    W2b = W2_ribo.astype(bf16)
    Wg1_bf = Wg1.astype(bf16)
    Wg2_bf = Wg2.astype(bf16)
    Wdr_bf = Wd_rna.astype(bf16)
    Wdb_bf = Wd_ribo.astype(bf16)

    pad = E_PAD - E
    col3 = jnp.concatenate(
        [edge_index[1], jnp.zeros((pad,), jnp.int32)]).reshape(TOTAL_CHUNKS, CHUNK)
    row3 = jnp.concatenate(
        [edge_index[0], jnp.zeros((pad,), jnp.int32)]).reshape(TOTAL_CHUNKS, CHUNK)
    w3 = jnp.concatenate(
        [edge_weight, jnp.zeros((pad,), f32)]).reshape(TOTAL_CHUNKS, CHUNK)

    y1 = _run_encoder(rna_bf, ribo_bf, W1r, b1_rna, g1_rna, be1_rna, W2r,
                      b2_rna, W1b, b1_ribo, g1_ribo, be1_ribo, W2b, b2_ribo,
                      Wf, bf, Wg1_bf)
    zeros2d = jnp.zeros((CHUNK, DIM), f32)
    s1 = _spmm(y1, col3, row3, w3, zeros2d)
    y2 = _run_gcn1(s1, bg1, Wg2_bf)
    s2 = _spmm(y2, col3, row3, w3, zeros2d)
    h = _run_bias(s2, bg2)
    s3 = _spmm(h, col3, row3, w3, zeros2d)
    recon_rna, recon_ribo = _run_decoder(s3, Wdr_bf, Wdb_bf)
    return (h, recon_rna, recon_ribo)


# speedup vs baseline: 1.4064x; 1.0283x over previous
import dataclasses
import functools

import jax
import jax.numpy as jnp
from jax import lax
from jax.experimental import pallas as pl
from jax.experimental.pallas import tpu as pltpu
from jax.experimental.pallas import tpu_sc as plsc

N = 10000
N_GENES = 2048
DIM = 128
HID = 512
E = 160000

NC = 2
NS = 16
LANES = 16
NW = NC * NS
CHUNK = 80
E_PAD = 163840
TOTAL_CHUNKS = E_PAD // CHUNK
C0 = 120
C1 = 8
NBUF = 3
SLAB = 200
NSLAB = N // SLAB

BLK_A = 1000
BLK_E = 1000
BLK_C = 1000


def _erf(x):
    return lax.erf(x)


def _encoder_block(x_bf, W1, b1, g, be, W2):
    h = jnp.dot(x_bf, W1, preferred_element_type=jnp.float32) + b1
    mu = jnp.mean(h, axis=-1, keepdims=True)
    var = jnp.mean((h - mu) ** 2, axis=-1, keepdims=True)
    h = (h - mu) * lax.rsqrt(var + 1e-5) * g + be
    h = 0.5 * h * (1.0 + _erf(h * 0.7071067811865476))
    return jnp.dot(h.astype(jnp.bfloat16), W2, preferred_element_type=jnp.float32)


def _enc_kernel(rna_ref, ribo_ref, W1r_ref, b1r_ref, g1r_ref, be1r_ref,
                W2r_ref, b2r_ref, W1b_ref, b1b_ref, g1b_ref, be1b_ref,
                W2b_ref, b2b_ref, Wf_ref, bf_ref, Wg1_ref, out_ref):
    z_rna = _encoder_block(rna_ref[...], W1r_ref[...], b1r_ref[...],
                           g1r_ref[...], be1r_ref[...], W2r_ref[...]) + b2r_ref[...]
    z_ribo = _encoder_block(ribo_ref[...], W1b_ref[...], b1b_ref[...],
                            g1b_ref[...], be1b_ref[...], W2b_ref[...]) + b2b_ref[...]
    wf = Wf_ref[...].reshape(1, DIM)
    s_rna = jnp.sum(jnp.tanh(z_rna) * wf, axis=-1, keepdims=True) + bf_ref[...]
    s_ribo = jnp.sum(jnp.tanh(z_ribo) * wf, axis=-1, keepdims=True) + bf_ref[...]
    m = jnp.maximum(s_rna, s_ribo)
    e_rna = jnp.exp(s_rna - m)
    e_ribo = jnp.exp(s_ribo - m)
    denom = e_rna + e_ribo
    z = (e_rna * z_rna + e_ribo * z_ribo) / denom
    out_ref[...] = jnp.dot(z.astype(jnp.bfloat16), Wg1_ref[...],
                           preferred_element_type=jnp.float32)


def _run_encoder(rna_bf, ribo_bf, W1r, b1r, g1r, be1r, W2r, b2r,
                 W1b, b1b, g1b, be1b, W2b, b2b, Wf, bf, Wg1):
    nblk = N // BLK_A
    row_spec = pl.BlockSpec((BLK_A, N_GENES), lambda i: (i, 0))
    full = lambda arr: pl.BlockSpec(arr.shape, lambda i: (0,) * arr.ndim)
    return pl.pallas_call(
        _enc_kernel,
        grid=(nblk,),
        in_specs=[row_spec, row_spec] + [full(a) for a in (
            W1r, b1r, g1r, be1r, W2r, b2r, W1b, b1b, g1b, be1b, W2b, b2b,
            Wf, bf, Wg1)],
        out_specs=pl.BlockSpec((BLK_A, DIM), lambda i: (i, 0)),
        out_shape=jax.ShapeDtypeStruct((N, DIM), jnp.float32),
    )(rna_bf, ribo_bf, W1r, b1r, g1r, be1r, W2r, b2r,
      W1b, b1b, g1b, be1b, W2b, b2b, Wf, bf, Wg1)


def _gcn1_kernel(p_ref, bg1_ref, Wg2_ref, out_ref):
    t = jax.nn.relu(p_ref[0] + p_ref[1] + bg1_ref[...])
    out_ref[...] = jnp.dot(t.astype(jnp.bfloat16), Wg2_ref[...],
                           preferred_element_type=jnp.float32)


def _run_gcn1(partials, bg1, Wg2_bf):
    nblk = N // BLK_C
    return pl.pallas_call(
        _gcn1_kernel,
        grid=(nblk,),
        in_specs=[pl.BlockSpec((2, BLK_C, DIM), lambda i: (0, i, 0)),
                  pl.BlockSpec(bg1.shape, lambda i: (0,)),
                  pl.BlockSpec(Wg2_bf.shape, lambda i: (0, 0))],
        out_specs=pl.BlockSpec((BLK_C, DIM), lambda i: (i, 0)),
        out_shape=jax.ShapeDtypeStruct((N, DIM), jnp.float32),
    )(partials, bg1, Wg2_bf)


def _bias_kernel(p_ref, b_ref, out_ref):
    out_ref[...] = p_ref[0] + p_ref[1] + b_ref[...]


def _run_bias(partials, b):
    nblk = N // BLK_C
    return pl.pallas_call(
        _bias_kernel,
        grid=(nblk,),
        in_specs=[pl.BlockSpec((2, BLK_C, DIM), lambda i: (0, i, 0)),
                  pl.BlockSpec(b.shape, lambda i: (0,))],
        out_specs=pl.BlockSpec((BLK_C, DIM), lambda i: (i, 0)),
        out_shape=jax.ShapeDtypeStruct((N, DIM), jnp.float32),
    )(partials, b)


def _dec_kernel(p_ref, Wdr_ref, Wdb_ref, rna_ref, ribo_ref):
    ah = (p_ref[0] + p_ref[1]).astype(jnp.bfloat16)
    rna_ref[...] = jnp.dot(ah, Wdr_ref[...], preferred_element_type=jnp.float32)
    ribo_ref[...] = jnp.dot(ah, Wdb_ref[...], preferred_element_type=jnp.float32)


def _run_decoder(partials, Wdr_bf, Wdb_bf):
    nblk = N // BLK_E
    return pl.pallas_call(
        _dec_kernel,
        grid=(nblk,),
        in_specs=[pl.BlockSpec((2, BLK_E, DIM), lambda i: (0, i, 0)),
                  pl.BlockSpec(Wdr_bf.shape, lambda i: (0, 0)),
                  pl.BlockSpec(Wdb_bf.shape, lambda i: (0, 0))],
        out_specs=[pl.BlockSpec((BLK_E, N_GENES), lambda i: (i, 0)),
                   pl.BlockSpec((BLK_E, N_GENES), lambda i: (i, 0))],
        out_shape=[jax.ShapeDtypeStruct((N, N_GENES), jnp.float32),
                   jax.ShapeDtypeStruct((N, N_GENES), jnp.float32)],
    )(partials, Wdr_bf, Wdb_bf)


def _spmm_body(x_hbm, col_hbm, row_hbm, w_hbm, z_hbm, out_hbm, accum,
               col_v, wb0, wb1, wb2, rows0, rows1, rows2, ridx0, ridx1, ridx2,
               sg0, sg1, sg2, sr0, sr1, sr2, ss0, ss1, ss2):
    c = lax.axis_index("c")
    s = lax.axis_index("s")
    bufs = (rows0, rows1, rows2)
    wbuf = (wb0, wb1, wb2)
    ridx = (ridx0, ridx1, ridx2)
    sg = (sg0, sg1, sg2)
    sr = (sr0, sr1, sr2)
    ss = (ss0, ss1, ss2)

    def pipeline(base, nchunks):
        if nchunks == 0:
            plsc.subcore_barrier()
            return
        pltpu.sync_copy(col_hbm.at[pl.ds(base, nchunks)],
                        col_v.at[pl.ds(0, nchunks)])

        def issue(jj, b):
            pltpu.async_copy(row_hbm.at[base + jj], ridx[b], sr[b])
            pltpu.async_copy(w_hbm.at[base + jj], wbuf[b], sr[b])
            pltpu.async_copy(x_hbm.at[col_v.at[jj]], bufs[b], sg[b])

        def wait_issue(jj, b):
            pltpu.make_async_copy(row_hbm.at[base + jj], ridx[b], sr[b]).wait()
            pltpu.make_async_copy(w_hbm.at[base + jj], wbuf[b], sr[b]).wait()
            pltpu.make_async_copy(x_hbm.at[col_v.at[jj]], bufs[b], sg[b]).wait()

        def scale(jj, b):
            buf = bufs[b]

            @pl.loop(0, CHUNK, step=4)
            def _edge(l0):
                for dl in range(4):
                    lv = jnp.full((LANES,), l0 + dl, jnp.int32)
                    wb = plsc.load_gather(wbuf[b], [lv])
                    for q in range(DIM // LANES):
                        sl = pl.ds(q * LANES, LANES)
                        buf[l0 + dl, sl] = buf[l0 + dl, sl] * wb

        def section(jp, b, refill=True):
            jj = jp * NBUF + b
            wait_issue(jj, b)
            scale(jj, b)
            pltpu.async_copy(bufs[b], accum.at[ridx[b]], ss[b], add=True)
            if refill:
                nxt = jj + NBUF

                @pl.when(nxt < nchunks)
                def _():
                    pltpu.make_async_copy(bufs[b], accum.at[ridx[b]],
                                          ss[b]).wait()
                    issue(nxt, b)

        for b in range(NBUF):
            issue(b, b)

        nslab = N // CHUNK
        for k in range((nslab + NS - 1) // NS):
            ch = s + k * NS

            @pl.when(ch < nslab)
            def _():
                off = pl.multiple_of(ch * CHUNK, 8)
                pltpu.sync_copy(z_hbm, accum.at[pl.ds(off, CHUNK)])
        plsc.subcore_barrier()

        @pl.loop(0, nchunks // NBUF)
        def _pipe(jp):
            for b in range(NBUF):
                section(jp, b)

        for b in range(nchunks % NBUF):
            section(nchunks // NBUF, b, refill=False)
        for b in range(NBUF):
            pltpu.make_async_copy(bufs[b], accum.at[ridx[b]], ss[b]).wait()

    @pl.when(c == 0)
    def _():
        pipeline(s * C0, C0)

    @pl.when(c == 1)
    def _():
        pipeline(NS * C0 + s * C1, C1)

    plsc.subcore_barrier()

    for k in range((NSLAB + NS - 1) // NS):
        slab = s + k * NS

        @pl.when(slab < NSLAB)
        def _():
            off = pl.multiple_of(slab * SLAB, 8)
            pltpu.sync_copy(accum.at[pl.ds(off, SLAB)],
                            out_hbm.at[c, pl.ds(off, SLAB)])


@jax.jit
def _spmm(x, col3, row3, w3, zeros2d):
    mesh = plsc.VectorSubcoreMesh(core_axis_name="c", subcore_axis_name="s")
    cp = pltpu.CompilerParams()
    if "needs_layout_passes" in pltpu.CompilerParams.__dataclass_fields__:
        cp = dataclasses.replace(cp, needs_layout_passes=False)
    kern = pl.kernel(
        _spmm_body,
        out_type=jax.ShapeDtypeStruct((NC, N, DIM), jnp.float32),
        mesh=mesh,
        scratch_types=[
            pltpu.VMEM_SHARED((N, DIM), jnp.float32),
            pltpu.VMEM((C0, CHUNK), jnp.int32),
            pltpu.VMEM((CHUNK,), jnp.float32),
            pltpu.VMEM((CHUNK,), jnp.float32),
            pltpu.VMEM((CHUNK,), jnp.float32),
            pltpu.VMEM((CHUNK, DIM), jnp.float32),
            pltpu.VMEM((CHUNK, DIM), jnp.float32),
            pltpu.VMEM((CHUNK, DIM), jnp.float32),
            pltpu.VMEM((CHUNK,), jnp.int32),
            pltpu.VMEM((CHUNK,), jnp.int32),
            pltpu.VMEM((CHUNK,), jnp.int32),
            pltpu.SemaphoreType.DMA,
            pltpu.SemaphoreType.DMA,
            pltpu.SemaphoreType.DMA,
            pltpu.SemaphoreType.DMA,
            pltpu.SemaphoreType.DMA,
            pltpu.SemaphoreType.DMA,
            pltpu.SemaphoreType.DMA,
            pltpu.SemaphoreType.DMA,
            pltpu.SemaphoreType.DMA,
        ],
        compiler_params=cp,
    )
    return kern(x, col3, row3, w3, zeros2d)


def kernel(rna_expr, ribo_expr, edge_index, edge_weight, W1_rna, b1_rna,
           g1_rna, be1_rna, W2_rna, b2_rna, W1_ribo, b1_ribo, g1_ribo,
           be1_ribo, W2_ribo, b2_ribo, Wf, bf, Wg1, bg1, Wg2, bg2,
           Wd_rna, Wd_ribo):
    f32 = jnp.float32
    bf16 = jnp.bfloat16

    rna_bf = rna_expr.astype(bf16)
    ribo_bf = ribo_expr.astype(bf16)
    W1r = W1_rna.astype(bf16)
    W2r = W2_rna.astype(bf16)
    W1b = W1_ribo.astype(bf16)
    W2b = W2_ribo.astype(bf16)
    Wg1_bf = Wg1.astype(bf16)
    Wg2_bf = Wg2.astype(bf16)
    Wdr_bf = Wd_rna.astype(bf16)
    Wdb_bf = Wd_ribo.astype(bf16)

    pad = E_PAD - E
    col3 = jnp.concatenate(
        [edge_index[1], jnp.zeros((pad,), jnp.int32)]).reshape(TOTAL_CHUNKS, CHUNK)
    row3 = jnp.concatenate(
        [edge_index[0], jnp.zeros((pad,), jnp.int32)]).reshape(TOTAL_CHUNKS, CHUNK)
    w3 = jnp.concatenate(
        [edge_weight, jnp.zeros((pad,), f32)]).reshape(TOTAL_CHUNKS, CHUNK)

    y1 = _run_encoder(rna_bf, ribo_bf, W1r, b1_rna, g1_rna, be1_rna, W2r,
                      b2_rna, W1b, b1_ribo, g1_ribo, be1_ribo, W2b, b2_ribo,
                      Wf, bf, Wg1_bf)
    zeros2d = jnp.zeros((CHUNK, DIM), f32)
    s1 = _spmm(y1, col3, row3, w3, zeros2d)
    y2 = _run_gcn1(s1, bg1, Wg2_bf)
    s2 = _spmm(y2, col3, row3, w3, zeros2d)
    h = _run_bias(s2, bg2)
    s3 = _spmm(h, col3, row3, w3, zeros2d)
    recon_rna, recon_ribo = _run_decoder(s3, Wdr_bf, Wdb_bf)
    return (h, recon_rna, recon_ribo)


# speedup vs baseline: 1.4112x; 1.0034x over previous
import dataclasses

import jax
import jax.numpy as jnp
from jax import lax
from jax.experimental import pallas as pl
from jax.experimental.pallas import tpu as pltpu
from jax.experimental.pallas import tpu_sc as plsc

N = 10000
N_GENES = 2048
DIM = 128
HID = 512
E = 160000

NC = 2
NS = 16
LANES = 16
NW = NC * NS
CHUNK = 80
E_PAD = 163840
TOTAL_CHUNKS = E_PAD // CHUNK
C0 = 120
C1 = 8
NBUF = 3
SLAB = 200
NSLAB = N // SLAB

BLK_A = 1000
BLK_E = 1000
BLK_C = 1000


def _erf(x):
    return lax.erf(x)


def _encoder_block(x_bf, W1, b1, g, be, W2):
    h = jnp.dot(x_bf, W1, preferred_element_type=jnp.float32) + b1
    mu = jnp.mean(h, axis=-1, keepdims=True)
    var = jnp.mean((h - mu) ** 2, axis=-1, keepdims=True)
    h = (h - mu) * lax.rsqrt(var + 1e-5) * g + be
    h = 0.5 * h * (1.0 + _erf(h * 0.7071067811865476))
    return jnp.dot(h.astype(jnp.bfloat16), W2, preferred_element_type=jnp.float32)


def _enc_kernel(rna_ref, ribo_ref, W1r_ref, b1r_ref, g1r_ref, be1r_ref,
                W2r_ref, b2r_ref, W1b_ref, b1b_ref, g1b_ref, be1b_ref,
                W2b_ref, b2b_ref, Wf_ref, bf_ref, Wg1_ref, out_ref):
    z_rna = _encoder_block(rna_ref[...], W1r_ref[...], b1r_ref[...],
                           g1r_ref[...], be1r_ref[...], W2r_ref[...]) + b2r_ref[...]
    z_ribo = _encoder_block(ribo_ref[...], W1b_ref[...], b1b_ref[...],
                            g1b_ref[...], be1b_ref[...], W2b_ref[...]) + b2b_ref[...]
    wf = Wf_ref[...].reshape(1, DIM)
    s_rna = jnp.sum(jnp.tanh(z_rna) * wf, axis=-1, keepdims=True) + bf_ref[...]
    s_ribo = jnp.sum(jnp.tanh(z_ribo) * wf, axis=-1, keepdims=True) + bf_ref[...]
    m = jnp.maximum(s_rna, s_ribo)
    e_rna = jnp.exp(s_rna - m)
    e_ribo = jnp.exp(s_ribo - m)
    denom = e_rna + e_ribo
    z = (e_rna * z_rna + e_ribo * z_ribo) / denom
    out_ref[...] = jnp.dot(z.astype(jnp.bfloat16), Wg1_ref[...],
                           preferred_element_type=jnp.float32)


def _run_encoder(rna_bf, ribo_bf, W1r, b1r, g1r, be1r, W2r, b2r,
                 W1b, b1b, g1b, be1b, W2b, b2b, Wf, bf, Wg1):
    nblk = N // BLK_A
    row_spec = pl.BlockSpec((BLK_A, N_GENES), lambda i: (i, 0))
    full = lambda arr: pl.BlockSpec(arr.shape, lambda i: (0,) * arr.ndim)
    return pl.pallas_call(
        _enc_kernel,
        grid=(nblk,),
        in_specs=[row_spec, row_spec] + [full(a) for a in (
            W1r, b1r, g1r, be1r, W2r, b2r, W1b, b1b, g1b, be1b, W2b, b2b,
            Wf, bf, Wg1)],
        out_specs=pl.BlockSpec((BLK_A, DIM), lambda i: (i, 0)),
        out_shape=jax.ShapeDtypeStruct((N, DIM), jnp.float32),
    )(rna_bf, ribo_bf, W1r, b1r, g1r, be1r, W2r, b2r,
      W1b, b1b, g1b, be1b, W2b, b2b, Wf, bf, Wg1)


def _gcn1_kernel(p_ref, bg1_ref, Wg2_ref, out_ref):
    t = jax.nn.relu(p_ref[0] + p_ref[1] + bg1_ref[...])
    out_ref[...] = jnp.dot(t.astype(jnp.bfloat16), Wg2_ref[...],
                           preferred_element_type=jnp.float32)


def _run_gcn1(partials, bg1, Wg2_bf):
    nblk = N // BLK_C
    return pl.pallas_call(
        _gcn1_kernel,
        grid=(nblk,),
        in_specs=[pl.BlockSpec((2, BLK_C, DIM), lambda i: (0, i, 0)),
                  pl.BlockSpec(bg1.shape, lambda i: (0,)),
                  pl.BlockSpec(Wg2_bf.shape, lambda i: (0, 0))],
        out_specs=pl.BlockSpec((BLK_C, DIM), lambda i: (i, 0)),
        out_shape=jax.ShapeDtypeStruct((N, DIM), jnp.float32),
    )(partials, bg1, Wg2_bf)


def _bias_kernel(p_ref, b_ref, out_ref):
    out_ref[...] = p_ref[0] + p_ref[1] + b_ref[...]


def _run_bias(partials, b):
    nblk = N // BLK_C
    return pl.pallas_call(
        _bias_kernel,
        grid=(nblk,),
        in_specs=[pl.BlockSpec((2, BLK_C, DIM), lambda i: (0, i, 0)),
                  pl.BlockSpec(b.shape, lambda i: (0,))],
        out_specs=pl.BlockSpec((BLK_C, DIM), lambda i: (i, 0)),
        out_shape=jax.ShapeDtypeStruct((N, DIM), jnp.float32),
    )(partials, b)


def _dec_kernel(p_ref, Wdr_ref, Wdb_ref, rna_ref, ribo_ref):
    ah = (p_ref[0] + p_ref[1]).astype(jnp.bfloat16)
    rna_ref[...] = jnp.dot(ah, Wdr_ref[...], preferred_element_type=jnp.float32)
    ribo_ref[...] = jnp.dot(ah, Wdb_ref[...], preferred_element_type=jnp.float32)


def _run_decoder(partials, Wdr_bf, Wdb_bf):
    nblk = N // BLK_E
    return pl.pallas_call(
        _dec_kernel,
        grid=(nblk,),
        in_specs=[pl.BlockSpec((2, BLK_E, DIM), lambda i: (0, i, 0)),
                  pl.BlockSpec(Wdr_bf.shape, lambda i: (0, 0)),
                  pl.BlockSpec(Wdb_bf.shape, lambda i: (0, 0))],
        out_specs=[pl.BlockSpec((BLK_E, N_GENES), lambda i: (i, 0)),
                   pl.BlockSpec((BLK_E, N_GENES), lambda i: (i, 0))],
        out_shape=[jax.ShapeDtypeStruct((N, N_GENES), jnp.float32),
                   jax.ShapeDtypeStruct((N, N_GENES), jnp.float32)],
    )(partials, Wdr_bf, Wdb_bf)


def _spmm_body(x_hbm, col_hbm, row_hbm, w_hbm, z_hbm, out_hbm, accum,
               col_v, wb0, wb1, wb2, rows0, rows1, rows2, ridx0, ridx1, ridx2,
               sg0, sg1, sg2, sr0, sr1, sr2, ss0, ss1, ss2):
    c = lax.axis_index("c")
    s = lax.axis_index("s")
    bufs = (rows0, rows1, rows2)
    wbuf = (wb0, wb1, wb2)
    ridx = (ridx0, ridx1, ridx2)
    sg = (sg0, sg1, sg2)
    sr = (sr0, sr1, sr2)
    ss = (ss0, ss1, ss2)

    def pipeline(base, nchunks):
        if nchunks == 0:
            plsc.subcore_barrier()
            return
        pltpu.sync_copy(col_hbm.at[pl.ds(base, nchunks)],
                        col_v.at[pl.ds(0, nchunks)])

        def issue(jj, b):
            pltpu.async_copy(row_hbm.at[base + jj], ridx[b], sr[b])
            pltpu.async_copy(w_hbm.at[base + jj], wbuf[b], sr[b])
            pltpu.async_copy(x_hbm.at[col_v.at[jj]], bufs[b], sg[b])

        def wait_issue(jj, b):
            pltpu.make_async_copy(row_hbm.at[base + jj], ridx[b], sr[b]).wait()
            pltpu.make_async_copy(w_hbm.at[base + jj], wbuf[b], sr[b]).wait()
            pltpu.make_async_copy(x_hbm.at[col_v.at[jj]], bufs[b], sg[b]).wait()

        def scale(jj, b):
            buf = bufs[b]

            @pl.loop(0, CHUNK, step=4)
            def _edge(l0):
                for dl in range(4):
                    lv = jnp.full((LANES,), l0 + dl, jnp.int32)
                    wb = plsc.load_gather(wbuf[b], [lv])
                    for q in range(DIM // LANES):
                        sl = pl.ds(q * LANES, LANES)
                        buf[l0 + dl, sl] = buf[l0 + dl, sl] * wb

        def section(jp, b, refill=True):
            jj = jp * NBUF + b
            wait_issue(jj, b)
            scale(jj, b)
            pltpu.async_copy(bufs[b], accum.at[ridx[b]], ss[b], add=True)
            if refill:
                nxt = jj + NBUF

                @pl.when(nxt < nchunks)
                def _():
                    pltpu.make_async_copy(bufs[b], accum.at[ridx[b]],
                                          ss[b]).wait()
                    issue(nxt, b)

        for b in range(NBUF):
            issue(b, b)

        nslab = N // CHUNK
        for k in range((nslab + NS - 1) // NS):
            ch = s + k * NS

            @pl.when(ch < nslab)
            def _():
                off = pl.multiple_of(ch * CHUNK, 8)
                pltpu.sync_copy(z_hbm, accum.at[pl.ds(off, CHUNK)])
        plsc.subcore_barrier()

        @pl.loop(0, nchunks // NBUF)
        def _pipe(jp):
            for b in range(NBUF):
                section(jp, b)

        for b in range(nchunks % NBUF):
            section(nchunks // NBUF, b, refill=False)
        for b in range(NBUF):
            pltpu.make_async_copy(bufs[b], accum.at[ridx[b]], ss[b]).wait()

    @pl.when(c == 0)
    def _():
        pipeline(s * C0, C0)

    @pl.when(c == 1)
    def _():
        pipeline(NS * C0 + s * C1, C1)

    plsc.subcore_barrier()

    for k in range((NSLAB + NS - 1) // NS):
        slab = s + k * NS

        @pl.when(slab < NSLAB)
        def _():
            off = pl.multiple_of(slab * SLAB, 8)
            pltpu.sync_copy(accum.at[pl.ds(off, SLAB)],
                            out_hbm.at[c, pl.ds(off, SLAB)])


@jax.jit
def _spmm(x, col3, row3, w3, zeros2d):
    mesh = plsc.VectorSubcoreMesh(core_axis_name="c", subcore_axis_name="s")
    cp = pltpu.CompilerParams()
    if "needs_layout_passes" in pltpu.CompilerParams.__dataclass_fields__:
        cp = dataclasses.replace(cp, needs_layout_passes=False)
    kern = pl.kernel(
        _spmm_body,
        out_type=jax.ShapeDtypeStruct((NC, N, DIM), jnp.float32),
        mesh=mesh,
        scratch_types=[
            pltpu.VMEM_SHARED((N, DIM), jnp.float32),
            pltpu.VMEM((C0, CHUNK), jnp.int32),
            pltpu.VMEM((CHUNK,), jnp.float32),
            pltpu.VMEM((CHUNK,), jnp.float32),
            pltpu.VMEM((CHUNK,), jnp.float32),
            pltpu.VMEM((CHUNK, DIM), jnp.float32),
            pltpu.VMEM((CHUNK, DIM), jnp.float32),
            pltpu.VMEM((CHUNK, DIM), jnp.float32),
            pltpu.VMEM((CHUNK,), jnp.int32),
            pltpu.VMEM((CHUNK,), jnp.int32),
            pltpu.VMEM((CHUNK,), jnp.int32),
            pltpu.SemaphoreType.DMA,
            pltpu.SemaphoreType.DMA,
            pltpu.SemaphoreType.DMA,
            pltpu.SemaphoreType.DMA,
            pltpu.SemaphoreType.DMA,
            pltpu.SemaphoreType.DMA,
            pltpu.SemaphoreType.DMA,
            pltpu.SemaphoreType.DMA,
            pltpu.SemaphoreType.DMA,
        ],
        compiler_params=cp,
    )
    return kern(x, col3, row3, w3, zeros2d)


def kernel(rna_expr, ribo_expr, edge_index, edge_weight, W1_rna, b1_rna,
           g1_rna, be1_rna, W2_rna, b2_rna, W1_ribo, b1_ribo, g1_ribo,
           be1_ribo, W2_ribo, b2_ribo, Wf, bf, Wg1, bg1, Wg2, bg2,
           Wd_rna, Wd_ribo):
    f32 = jnp.float32
    bf16 = jnp.bfloat16

    rna_bf = rna_expr.astype(bf16)
    ribo_bf = ribo_expr.astype(bf16)
    W1r = W1_rna.astype(bf16)
    W2r = W2_rna.astype(bf16)
    W1b = W1_ribo.astype(bf16)
    W2b = W2_ribo.astype(bf16)
    Wg1_bf = Wg1.astype(bf16)
    Wg2_bf = Wg2.astype(bf16)
    Wdr_bf = Wd_rna.astype(bf16)
    Wdb_bf = Wd_ribo.astype(bf16)

    pad = E_PAD - E
    col3 = jnp.concatenate(
        [edge_index[1], jnp.zeros((pad,), jnp.int32)]).reshape(TOTAL_CHUNKS, CHUNK)
    row3 = jnp.concatenate(
        [edge_index[0], jnp.zeros((pad,), jnp.int32)]).reshape(TOTAL_CHUNKS, CHUNK)
    w3 = jnp.concatenate(
        [edge_weight, jnp.zeros((pad,), f32)]).reshape(TOTAL_CHUNKS, CHUNK)

    y1 = _run_encoder(rna_bf, ribo_bf, W1r, b1_rna, g1_rna, be1_rna, W2r,
                      b2_rna, W1b, b1_ribo, g1_ribo, be1_ribo, W2b, b2_ribo,
                      Wf, bf, Wg1_bf)
    zeros2d = jnp.zeros((CHUNK, DIM), f32)
    s1 = _spmm(y1, col3, row3, w3, zeros2d)
    y2 = _run_gcn1(s1, bg1, Wg2_bf)
    s2 = _spmm(y2, col3, row3, w3, zeros2d)
    h = _run_bias(s2, bg2)
    s3 = _spmm(h, col3, row3, w3, zeros2d)
    recon_rna, recon_ribo = _run_decoder(s3, Wdr_bf, Wdb_bf)
    return (h, recon_rna, recon_ribo)
